# baseline (speedup 1.0000x reference)
import functools
import math

import jax
import jax.numpy as jnp
from jax import lax
from jax.experimental import pallas as pl
from jax.experimental.pallas import tpu as pltpu
from jax.experimental.pallas import tpu_sc as plsc

MOMENTUM = 0.5

_NC, _NS = 2, 16
_NW = _NC * _NS
_CHUNK = 128


def _row_normalize(vmem_rows, n_rows, D):
    nv = D // 16

    def body(r, _):
        acc = jnp.zeros((16,), jnp.float32)
        for l in range(nv):
            v = vmem_rows[r, pl.ds(l * 16, 16)]
            acc = acc + v * v
        ss = jnp.sum(acc)
        ssv = jnp.full((16,), ss, jnp.float32)
        y = plsc.bitcast(
            jnp.int32(0x5F3759DF) - (plsc.bitcast(ssv, jnp.int32) >> 1),
            jnp.float32,
        )
        for _i in range(3):
            y = y * (1.5 - 0.5 * ssv * y * y)
        for l in range(nv):
            vmem_rows[r, pl.ds(l * 16, 16)] = vmem_rows[r, pl.ds(l * 16, 16)] * y
        return _

    lax.fori_loop(0, n_rows, body, None)


def _sc_upd(table_a, table_b, x1, x2, index, eff):
    N, D = table_a.shape
    B = index.shape[0]
    s_per_w = B // _NW
    nl = D // 16
    mesh = plsc.VectorSubcoreMesh(core_axis_name="c", subcore_axis_name="s")

    @functools.partial(
        pl.kernel,
        out_type=[
            jax.ShapeDtypeStruct((B, D), jnp.float32),
            jax.ShapeDtypeStruct((B, D), jnp.float32),
        ],
        mesh=mesh,
        scratch_types=[
            pltpu.VMEM((s_per_w,), jnp.int32),
            pltpu.VMEM((s_per_w,), jnp.int32),
            pltpu.VMEM((s_per_w, D), jnp.float32),
            pltpu.VMEM((s_per_w, D), jnp.float32),
            pltpu.SemaphoreType.DMA,
        ],
        compiler_params=pltpu.CompilerParams(needs_layout_passes=False),
    )
    def k(tab_a, tab_b, x1h, x2h, indexh, effh, upd1, upd2,
          pidx_v, peff_v, pos_v, x_v, psem):
        wid = lax.axis_index("s") * _NC + lax.axis_index("c")
        sbase = wid * s_per_w
        pltpu.sync_copy(indexh.at[pl.ds(sbase, s_per_w)], pidx_v)
        pltpu.sync_copy(effh.at[pl.ds(sbase, s_per_w)], peff_v)
        for mem_h, x_h, upd_h in ((tab_b, x1h, upd1), (tab_a, x2h, upd2)):
            pltpu.async_copy(mem_h.at[pidx_v], pos_v, psem).wait()
            pltpu.async_copy(x_h.at[peff_v], x_v, psem).wait()

            def ubody(r, _):
                for l in range(nl):
                    sl = pl.ds(l * 16, 16)
                    pos_v[r, sl] = pos_v[r, sl] * MOMENTUM + x_v[r, sl] * (1.0 - MOMENTUM)
                return _

            lax.fori_loop(0, s_per_w, ubody, None)
            _row_normalize(pos_v, s_per_w, D)
            pltpu.sync_copy(pos_v, upd_h.at[pl.ds(sbase, s_per_w)])

    return k(table_a, table_b, x1, x2, index, eff)


def _sc_fused(table_a, table_b, idx_flat, x1, x2):
    R = idx_flat.shape[0]
    N, D = table_a.shape
    B = x1.shape[0]
    K1 = R // B
    per_w = R // _NW
    n_units = per_w // _CHUNK
    s_per_w = B // _NW
    nl = D // 16
    mesh = plsc.VectorSubcoreMesh(core_axis_name="c", subcore_axis_name="s")

    @functools.partial(
        pl.kernel,
        out_type=[
            jax.ShapeDtypeStruct((R,), jnp.float32),
            jax.ShapeDtypeStruct((R,), jnp.float32),
        ],
        mesh=mesh,
        scratch_types=[
            pltpu.VMEM((2, _CHUNK), jnp.int32),
            pltpu.VMEM((2, _CHUNK, D), jnp.float32),
            pltpu.VMEM((2, _CHUNK, D), jnp.float32),
            pltpu.SemaphoreType.DMA((2,)),
            pltpu.SemaphoreType.DMA((2,)),
            pltpu.VMEM((per_w,), jnp.float32),
            pltpu.VMEM((per_w,), jnp.float32),
            pltpu.VMEM((s_per_w, D), jnp.float32),
            pltpu.VMEM((s_per_w, D), jnp.float32),
        ],
        compiler_params=pltpu.CompilerParams(needs_layout_passes=False),
    )
    def k(tab_a, tab_b, idxf, x1h, x2h,
          o1f, o2f,
          idx_v, rows_a, rows_b, sem_a, sem_b,
          l1, l2, xd1, xd2):
        wid = lax.axis_index("s") * _NC + lax.axis_index("c")
        base = wid * per_w
        sbase = wid * s_per_w

        pltpu.sync_copy(x1h.at[pl.ds(sbase, s_per_w)], xd1)
        pltpu.sync_copy(x2h.at[pl.ds(sbase, s_per_w)], xd2)

        lane15 = lax.iota(jnp.int32, 16) == 15

        def start(u, slot):
            off = base + u * _CHUNK
            pltpu.sync_copy(idxf.at[pl.ds(off, _CHUNK)], idx_v.at[slot])
            pltpu.async_copy(tab_a.at[idx_v.at[slot]], rows_a.at[slot], sem_a.at[slot])
            pltpu.async_copy(tab_b.at[idx_v.at[slot]], rows_b.at[slot], sem_b.at[slot])

        def compute(u, slot):
            pltpu.make_async_copy(tab_a.at[idx_v.at[slot]], rows_a.at[slot], sem_a.at[slot]).wait()
            pltpu.make_async_copy(tab_b.at[idx_v.at[slot]], rows_b.at[slot], sem_b.at[slot]).wait()
            s = u // (K1 // _CHUNK)
            lbase = u * _CHUNK
            xv1 = [xd1[s, pl.ds(16 * l, 16)] for l in range(nl)]
            xv2 = [xd2[s, pl.ds(16 * l, 16)] for l in range(nl)]

            def tree(v):
                while len(v) > 1:
                    v = [v[i] + v[i + 1] for i in range(0, len(v) - 1, 2)] + (
                        [v[-1]] if len(v) % 2 else [])
                return v[0]

            def jbody(t, _):
                for r4 in range(8):
                    j = t * 8 + r4
                    pa = [rows_a[slot, j, pl.ds(16 * l, 16)] * xv1[l] for l in range(nl)]
                    pb = [rows_b[slot, j, pl.ds(16 * l, 16)] * xv2[l] for l in range(nl)]
                    tgt = jnp.full((16,), lbase + j, jnp.int32)
                    plsc.store_scatter(l1, [tgt], plsc.cumsum(tree(pa)), mask=lane15)
                    plsc.store_scatter(l2, [tgt], plsc.cumsum(tree(pb)), mask=lane15)
                return _

            lax.fori_loop(0, _CHUNK // 8, jbody, None)

        start(0, 0)
        start(1, 1)

        def gbody(t, _):
            u0 = 2 * t
            compute(u0, 0)

            @pl.when(u0 + 2 < n_units)
            def _():
                start(u0 + 2, 0)

            compute(u0 + 1, 1)

            @pl.when(u0 + 3 < n_units)
            def _():
                start(u0 + 3, 1)

            return _

        lax.fori_loop(0, n_units // 2, gbody, None)

        ng = K1 // 16

        def smax(s, _):
            lb = s * K1
            v1 = [l1[pl.ds(lb + 16 * g, 16)] for g in range(ng)]
            m = v1[0]
            for g in range(1, ng):
                m = jnp.maximum(m, v1[g])
            mv = jnp.full((16,), jnp.max(m), jnp.float32)
            e1 = [jnp.exp(v - mv) for v in v1]
            ssum = e1[0]
            for g in range(1, ng):
                ssum = ssum + e1[g]
            sv = jnp.full((16,), jnp.sum(ssum), jnp.float32)
            o1 = [ev / sv for ev in e1]
            mo = o1[0]
            for g in range(1, ng):
                mo = jnp.maximum(mo, o1[g])
            for g in range(ng):
                l1[pl.ds(lb + 16 * g, 16)] = o1[g]
            mov = jnp.full((16,), jnp.max(mo), jnp.float32)
            v2 = [l2[pl.ds(lb + 16 * g, 16)] for g in range(ng)]
            e2 = [jnp.exp(v - mov) for v in v2]
            ssum2 = e2[0]
            for g in range(1, ng):
                ssum2 = ssum2 + e2[g]
            sv2 = jnp.full((16,), jnp.sum(ssum2), jnp.float32)
            for g in range(ng):
                l2[pl.ds(lb + 16 * g, 16)] = e2[g] / sv2
            return _

        lax.fori_loop(0, s_per_w, smax, None)

        pltpu.sync_copy(l1, o1f.at[pl.ds(base, per_w)])
        pltpu.sync_copy(l2, o2f.at[pl.ds(base, per_w)])

    return k(table_a, table_b, idx_flat, x1, x2)


_CP = 128000


def _copy_body(a_ref, b_ref, oa_ref, ob_ref):
    oa_ref[...] = a_ref[...]
    ob_ref[...] = b_ref[...]


def _tc_copy(a, b):
    E = a.shape[0]
    grid = (E // _CP,)
    return pl.pallas_call(
        _copy_body,
        grid=grid,
        in_specs=[
            pl.BlockSpec((_CP,), lambda i: (i,)),
            pl.BlockSpec((_CP,), lambda i: (i,)),
        ],
        out_specs=[
            pl.BlockSpec((_CP,), lambda i: (i,)),
            pl.BlockSpec((_CP,), lambda i: (i,)),
        ],
        out_shape=[
            jax.ShapeDtypeStruct((E,), jnp.float32),
            jax.ShapeDtypeStruct((E,), jnp.float32),
        ],
    )(a, b)


_TB = 32


def _scat_body(u1_ref, u2_ref, idxs_ref, cna_ref, cnb_ref, na_ref, nb_ref, sem):
    D = u1_ref.shape[0] // _TB
    descs = []
    for r in range(_TB):
        tgt = idxs_ref[0, 0, r]
        descs.append(pltpu.make_async_copy(
            u1_ref.at[pl.ds(r * D, D)], na_ref.at[pl.ds(tgt * D, D)], sem))
        descs.append(pltpu.make_async_copy(
            u2_ref.at[pl.ds(r * D, D)], nb_ref.at[pl.ds(tgt * D, D)], sem))
    for d in descs:
        d.start()
    for d in descs:
        d.wait()


def _tc_scatter(u1f, u2f, index3d, cna, cnb):
    B = index3d.shape[0] * _TB
    D = u1f.shape[0] // B
    E = cna.shape[0]
    grid = (B // _TB,)
    return pl.pallas_call(
        _scat_body,
        grid=grid,
        in_specs=[
            pl.BlockSpec((_TB * D,), lambda i: (i,)),
            pl.BlockSpec((_TB * D,), lambda i: (i,)),
            pl.BlockSpec((1, 1, _TB), lambda i: (i, 0, 0), memory_space=pltpu.SMEM),
            pl.BlockSpec(memory_space=pl.ANY),
            pl.BlockSpec(memory_space=pl.ANY),
        ],
        out_specs=[
            pl.BlockSpec(memory_space=pl.ANY),
            pl.BlockSpec(memory_space=pl.ANY),
        ],
        out_shape=[
            jax.ShapeDtypeStruct((E,), jnp.float32),
            jax.ShapeDtypeStruct((E,), jnp.float32),
        ],
        input_output_aliases={3: 0, 4: 1},
        scratch_shapes=[pltpu.SemaphoreType.DMA],
    )(u1f, u2f, index3d, cna, cnb)


def kernel(x1, x2, memory_x1, memory_x2, index, idx):
    B, D = x1.shape
    N = memory_x1.shape[0]
    K1 = idx.shape[1]
    flat = idx.reshape(-1)
    ar = jnp.arange(B, dtype=jnp.int32)
    eff = jnp.max(jnp.where(index[None, :] == index[:, None], ar[None, :], -1), axis=1)
    upd1, upd2 = _sc_upd(memory_x2, memory_x1, x1, x2, index, eff)
    o1f, o2f = _sc_fused(memory_x2, memory_x1, flat, x1, x2)
    cna, cnb = _tc_copy(memory_x1.reshape(-1), memory_x2.reshape(-1))
    na, nb = _tc_scatter(
        upd1.reshape(-1), upd2.reshape(-1), index.reshape(B // _TB, 1, _TB),
        cna, cnb)
    return (o1f.reshape(B, K1, 1), o2f.reshape(B, K1, 1),
            na.reshape(N, D), nb.reshape(N, D))


# speedup vs baseline: 1.2158x; 1.2158x over previous
import functools
import math

import jax
import jax.numpy as jnp
from jax import lax
from jax.experimental import pallas as pl
from jax.experimental.pallas import tpu as pltpu
from jax.experimental.pallas import tpu_sc as plsc

MOMENTUM = 0.5

_NC, _NS = 2, 16
_NW = _NC * _NS
_CHUNK = 128


def _row_normalize(vmem_rows, n_rows, D):
    nv = D // 16

    def body(r, _):
        acc = jnp.zeros((16,), jnp.float32)
        for l in range(nv):
            v = vmem_rows[r, pl.ds(l * 16, 16)]
            acc = acc + v * v
        ss = jnp.sum(acc)
        ssv = jnp.full((16,), ss, jnp.float32)
        y = plsc.bitcast(
            jnp.int32(0x5F3759DF) - (plsc.bitcast(ssv, jnp.int32) >> 1),
            jnp.float32,
        )
        for _i in range(3):
            y = y * (1.5 - 0.5 * ssv * y * y)
        for l in range(nv):
            vmem_rows[r, pl.ds(l * 16, 16)] = vmem_rows[r, pl.ds(l * 16, 16)] * y
        return _

    lax.fori_loop(0, n_rows, body, None)


def _sc_upd(table_a, table_b, x1, x2, index, eff):
    N, D = table_a.shape
    B = index.shape[0]
    s_per_w = B // _NW
    nl = D // 16
    mesh = plsc.VectorSubcoreMesh(core_axis_name="c", subcore_axis_name="s")

    @functools.partial(
        pl.kernel,
        out_type=[
            jax.ShapeDtypeStruct((B, D), jnp.float32),
            jax.ShapeDtypeStruct((B, D), jnp.float32),
        ],
        mesh=mesh,
        scratch_types=[
            pltpu.VMEM((s_per_w,), jnp.int32),
            pltpu.VMEM((s_per_w,), jnp.int32),
            pltpu.VMEM((s_per_w, D), jnp.float32),
            pltpu.VMEM((s_per_w, D), jnp.float32),
            pltpu.SemaphoreType.DMA,
        ],
        compiler_params=pltpu.CompilerParams(needs_layout_passes=False),
    )
    def k(tab_a, tab_b, x1h, x2h, indexh, effh, upd1, upd2,
          pidx_v, peff_v, pos_v, x_v, psem):
        wid = lax.axis_index("s") * _NC + lax.axis_index("c")
        sbase = wid * s_per_w
        pltpu.sync_copy(indexh.at[pl.ds(sbase, s_per_w)], pidx_v)
        pltpu.sync_copy(effh.at[pl.ds(sbase, s_per_w)], peff_v)
        for mem_h, x_h, upd_h in ((tab_b, x1h, upd1), (tab_a, x2h, upd2)):
            pltpu.async_copy(mem_h.at[pidx_v], pos_v, psem).wait()
            pltpu.async_copy(x_h.at[peff_v], x_v, psem).wait()

            def ubody(r, _):
                for l in range(nl):
                    sl = pl.ds(l * 16, 16)
                    pos_v[r, sl] = pos_v[r, sl] * MOMENTUM + x_v[r, sl] * (1.0 - MOMENTUM)
                return _

            lax.fori_loop(0, s_per_w, ubody, None)
            _row_normalize(pos_v, s_per_w, D)
            pltpu.sync_copy(pos_v, upd_h.at[pl.ds(sbase, s_per_w)])

    return k(table_a, table_b, x1, x2, index, eff)


def _sc_fused(table_a, table_b, idx_flat, x1, x2):
    R = idx_flat.shape[0]
    N, D = table_a.shape
    B = x1.shape[0]
    K1 = R // B
    per_w = R // _NW
    n_units = per_w // _CHUNK
    s_per_w = B // _NW
    nl = D // 16
    mesh = plsc.VectorSubcoreMesh(core_axis_name="c", subcore_axis_name="s")

    @functools.partial(
        pl.kernel,
        out_type=[
            jax.ShapeDtypeStruct((R,), jnp.float32),
            jax.ShapeDtypeStruct((R,), jnp.float32),
        ],
        mesh=mesh,
        scratch_types=[
            pltpu.VMEM((n_units, _CHUNK), jnp.int32),
            pltpu.VMEM((2, _CHUNK, D), jnp.float32),
            pltpu.VMEM((2, _CHUNK, D), jnp.float32),
            pltpu.SemaphoreType.DMA((2,)),
            pltpu.SemaphoreType.DMA((2,)),
            pltpu.VMEM((per_w,), jnp.float32),
            pltpu.VMEM((per_w,), jnp.float32),
            pltpu.VMEM((s_per_w, D), jnp.float32),
            pltpu.VMEM((s_per_w, D), jnp.float32),
        ],
        compiler_params=pltpu.CompilerParams(needs_layout_passes=False),
    )
    def k(tab_a, tab_b, idxf, x1h, x2h,
          o1f, o2f,
          idx_v, rows_a, rows_b, sem_a, sem_b,
          l1, l2, xd1, xd2):
        wid = lax.axis_index("s") * _NC + lax.axis_index("c")
        base = wid * per_w
        sbase = wid * s_per_w

        pltpu.sync_copy(x1h.at[pl.ds(sbase, s_per_w)], xd1)
        pltpu.sync_copy(x2h.at[pl.ds(sbase, s_per_w)], xd2)
        pltpu.sync_copy(idxf.at[pl.ds(wid * n_units, n_units)], idx_v)

        lane15 = lax.iota(jnp.int32, 16) == 15

        def start(u, slot):
            pltpu.async_copy(tab_a.at[idx_v.at[u]], rows_a.at[slot], sem_a.at[slot])
            pltpu.async_copy(tab_b.at[idx_v.at[u]], rows_b.at[slot], sem_b.at[slot])

        def compute(u, slot):
            pltpu.make_async_copy(tab_a.at[idx_v.at[u]], rows_a.at[slot], sem_a.at[slot]).wait()
            pltpu.make_async_copy(tab_b.at[idx_v.at[u]], rows_b.at[slot], sem_b.at[slot]).wait()
            s = u // (K1 // _CHUNK)
            lbase = u * _CHUNK
            xv1 = [xd1[s, pl.ds(16 * l, 16)] for l in range(nl)]
            xv2 = [xd2[s, pl.ds(16 * l, 16)] for l in range(nl)]

            def jbody(t, _):
                for r4 in range(8):
                    j = t * 8 + r4
                    acc_a = rows_a[slot, j, pl.ds(0, 16)] * xv1[0]
                    acc_b = rows_b[slot, j, pl.ds(0, 16)] * xv2[0]
                    for l in range(1, nl):
                        sl = pl.ds(16 * l, 16)
                        acc_a = acc_a + rows_a[slot, j, sl] * xv1[l]
                        acc_b = acc_b + rows_b[slot, j, sl] * xv2[l]
                    tgt = jnp.full((16,), lbase + j, jnp.int32)
                    plsc.store_scatter(l1, [tgt], plsc.cumsum(acc_a), mask=lane15)
                    plsc.store_scatter(l2, [tgt], plsc.cumsum(acc_b), mask=lane15)
                return _

            lax.fori_loop(0, _CHUNK // 8, jbody, None)

        start(0, 0)
        start(1, 1)

        def gbody(t, _):
            u0 = 2 * t
            compute(u0, 0)

            @pl.when(u0 + 2 < n_units)
            def _():
                start(u0 + 2, 0)

            compute(u0 + 1, 1)

            @pl.when(u0 + 3 < n_units)
            def _():
                start(u0 + 3, 1)

            return _

        lax.fori_loop(0, n_units // 2, gbody, None)

        ng = K1 // 16

        def smax(s, _):
            lb = s * K1
            v1 = [l1[pl.ds(lb + 16 * g, 16)] for g in range(ng)]
            m = v1[0]
            for g in range(1, ng):
                m = jnp.maximum(m, v1[g])
            mv = jnp.full((16,), jnp.max(m), jnp.float32)
            e1 = [jnp.exp(v - mv) for v in v1]
            ssum = e1[0]
            for g in range(1, ng):
                ssum = ssum + e1[g]
            sv = jnp.full((16,), jnp.sum(ssum), jnp.float32)
            o1 = [ev / sv for ev in e1]
            mo = o1[0]
            for g in range(1, ng):
                mo = jnp.maximum(mo, o1[g])
            for g in range(ng):
                l1[pl.ds(lb + 16 * g, 16)] = o1[g]
            mov = jnp.full((16,), jnp.max(mo), jnp.float32)
            v2 = [l2[pl.ds(lb + 16 * g, 16)] for g in range(ng)]
            e2 = [jnp.exp(v - mov) for v in v2]
            ssum2 = e2[0]
            for g in range(1, ng):
                ssum2 = ssum2 + e2[g]
            sv2 = jnp.full((16,), jnp.sum(ssum2), jnp.float32)
            for g in range(ng):
                l2[pl.ds(lb + 16 * g, 16)] = e2[g] / sv2
            return _

        lax.fori_loop(0, s_per_w, smax, None)

        pltpu.sync_copy(l1, o1f.at[pl.ds(base, per_w)])
        pltpu.sync_copy(l2, o2f.at[pl.ds(base, per_w)])

    return k(table_a, table_b, idx_flat.reshape(-1, _CHUNK), x1, x2)


_CP = 128000


def _copy_body(a_ref, b_ref, oa_ref, ob_ref):
    oa_ref[...] = a_ref[...]
    ob_ref[...] = b_ref[...]


def _tc_copy(a, b):
    E = a.shape[0]
    grid = (E // _CP,)
    return pl.pallas_call(
        _copy_body,
        grid=grid,
        in_specs=[
            pl.BlockSpec((_CP,), lambda i: (i,)),
            pl.BlockSpec((_CP,), lambda i: (i,)),
        ],
        out_specs=[
            pl.BlockSpec((_CP,), lambda i: (i,)),
            pl.BlockSpec((_CP,), lambda i: (i,)),
        ],
        out_shape=[
            jax.ShapeDtypeStruct((E,), jnp.float32),
            jax.ShapeDtypeStruct((E,), jnp.float32),
        ],
    )(a, b)


_TB = 32


def _scat_body(u1_ref, u2_ref, idxs_ref, cna_ref, cnb_ref, na_ref, nb_ref, sem):
    D = u1_ref.shape[0] // _TB
    descs = []
    for r in range(_TB):
        tgt = idxs_ref[0, 0, r]
        descs.append(pltpu.make_async_copy(
            u1_ref.at[pl.ds(r * D, D)], na_ref.at[pl.ds(tgt * D, D)], sem))
        descs.append(pltpu.make_async_copy(
            u2_ref.at[pl.ds(r * D, D)], nb_ref.at[pl.ds(tgt * D, D)], sem))
    for d in descs:
        d.start()
    for d in descs:
        d.wait()


def _tc_scatter(u1f, u2f, index3d, cna, cnb):
    B = index3d.shape[0] * _TB
    D = u1f.shape[0] // B
    E = cna.shape[0]
    grid = (B // _TB,)
    return pl.pallas_call(
        _scat_body,
        grid=grid,
        in_specs=[
            pl.BlockSpec((_TB * D,), lambda i: (i,)),
            pl.BlockSpec((_TB * D,), lambda i: (i,)),
            pl.BlockSpec((1, 1, _TB), lambda i: (i, 0, 0), memory_space=pltpu.SMEM),
            pl.BlockSpec(memory_space=pl.ANY),
            pl.BlockSpec(memory_space=pl.ANY),
        ],
        out_specs=[
            pl.BlockSpec(memory_space=pl.ANY),
            pl.BlockSpec(memory_space=pl.ANY),
        ],
        out_shape=[
            jax.ShapeDtypeStruct((E,), jnp.float32),
            jax.ShapeDtypeStruct((E,), jnp.float32),
        ],
        input_output_aliases={3: 0, 4: 1},
        scratch_shapes=[pltpu.SemaphoreType.DMA],
    )(u1f, u2f, index3d, cna, cnb)


def kernel(x1, x2, memory_x1, memory_x2, index, idx):
    B, D = x1.shape
    N = memory_x1.shape[0]
    K1 = idx.shape[1]
    flat = idx.reshape(-1)
    ar = jnp.arange(B, dtype=jnp.int32)
    eff = jnp.max(jnp.where(index[None, :] == index[:, None], ar[None, :], -1), axis=1)
    upd1, upd2 = _sc_upd(memory_x2, memory_x1, x1, x2, index, eff)
    o1f, o2f = _sc_fused(memory_x2, memory_x1, flat, x1, x2)
    cna, cnb = _tc_copy(memory_x1.reshape(-1), memory_x2.reshape(-1))
    na, nb = _tc_scatter(
        upd1.reshape(-1), upd2.reshape(-1), index.reshape(B // _TB, 1, _TB),
        cna, cnb)
    return (o1f.reshape(B, K1, 1), o2f.reshape(B, K1, 1),
            na.reshape(N, D), nb.reshape(N, D))


# speedup vs baseline: 1.2279x; 1.0099x over previous
import functools
import math

import jax
import jax.numpy as jnp
from jax import lax
from jax.experimental import pallas as pl
from jax.experimental.pallas import tpu as pltpu
from jax.experimental.pallas import tpu_sc as plsc

MOMENTUM = 0.5

_NC, _NS = 2, 16
_NW = _NC * _NS
_CHUNK = 64


def _row_normalize(vmem_rows, n_rows, D):
    nv = D // 16

    def body(r, _):
        acc = jnp.zeros((16,), jnp.float32)
        for l in range(nv):
            v = vmem_rows[r, pl.ds(l * 16, 16)]
            acc = acc + v * v
        ss = jnp.sum(acc)
        ssv = jnp.full((16,), ss, jnp.float32)
        y = plsc.bitcast(
            jnp.int32(0x5F3759DF) - (plsc.bitcast(ssv, jnp.int32) >> 1),
            jnp.float32,
        )
        for _i in range(3):
            y = y * (1.5 - 0.5 * ssv * y * y)
        for l in range(nv):
            vmem_rows[r, pl.ds(l * 16, 16)] = vmem_rows[r, pl.ds(l * 16, 16)] * y
        return _

    lax.fori_loop(0, n_rows, body, None)


def _sc_upd(table_a, table_b, x1, x2, index, eff):
    N, D = table_a.shape
    B = index.shape[0]
    s_per_w = B // _NW
    nl = D // 16
    mesh = plsc.VectorSubcoreMesh(core_axis_name="c", subcore_axis_name="s")

    @functools.partial(
        pl.kernel,
        out_type=[
            jax.ShapeDtypeStruct((B, D), jnp.float32),
            jax.ShapeDtypeStruct((B, D), jnp.float32),
        ],
        mesh=mesh,
        scratch_types=[
            pltpu.VMEM((s_per_w,), jnp.int32),
            pltpu.VMEM((s_per_w,), jnp.int32),
            pltpu.VMEM((s_per_w, D), jnp.float32),
            pltpu.VMEM((s_per_w, D), jnp.float32),
            pltpu.SemaphoreType.DMA,
        ],
        compiler_params=pltpu.CompilerParams(needs_layout_passes=False),
    )
    def k(tab_a, tab_b, x1h, x2h, indexh, effh, upd1, upd2,
          pidx_v, peff_v, pos_v, x_v, psem):
        wid = lax.axis_index("s") * _NC + lax.axis_index("c")
        sbase = wid * s_per_w
        pltpu.sync_copy(indexh.at[pl.ds(sbase, s_per_w)], pidx_v)
        pltpu.sync_copy(effh.at[pl.ds(sbase, s_per_w)], peff_v)
        for mem_h, x_h, upd_h in ((tab_b, x1h, upd1), (tab_a, x2h, upd2)):
            pltpu.async_copy(mem_h.at[pidx_v], pos_v, psem).wait()
            pltpu.async_copy(x_h.at[peff_v], x_v, psem).wait()

            def ubody(r, _):
                for l in range(nl):
                    sl = pl.ds(l * 16, 16)
                    pos_v[r, sl] = pos_v[r, sl] * MOMENTUM + x_v[r, sl] * (1.0 - MOMENTUM)
                return _

            lax.fori_loop(0, s_per_w, ubody, None)
            _row_normalize(pos_v, s_per_w, D)
            pltpu.sync_copy(pos_v, upd_h.at[pl.ds(sbase, s_per_w)])

    return k(table_a, table_b, x1, x2, index, eff)


def _sc_fused(table_a, table_b, idx_flat, x1, x2):
    R = idx_flat.shape[0]
    N, D = table_a.shape
    B = x1.shape[0]
    K1 = R // B
    per_w = R // _NW
    n_units = per_w // _CHUNK
    s_per_w = B // _NW
    nl = D // 16
    mesh = plsc.VectorSubcoreMesh(core_axis_name="c", subcore_axis_name="s")

    @functools.partial(
        pl.kernel,
        out_type=[
            jax.ShapeDtypeStruct((R,), jnp.float32),
            jax.ShapeDtypeStruct((R,), jnp.float32),
        ],
        mesh=mesh,
        scratch_types=[
            pltpu.VMEM((n_units, _CHUNK), jnp.int32),
            pltpu.VMEM((4, _CHUNK, D), jnp.float32),
            pltpu.VMEM((4, _CHUNK, D), jnp.float32),
            pltpu.SemaphoreType.DMA((4,)),
            pltpu.SemaphoreType.DMA((4,)),
            pltpu.VMEM((per_w,), jnp.float32),
            pltpu.VMEM((per_w,), jnp.float32),
            pltpu.VMEM((s_per_w, D), jnp.float32),
            pltpu.VMEM((s_per_w, D), jnp.float32),
        ],
        compiler_params=pltpu.CompilerParams(needs_layout_passes=False),
    )
    def k(tab_a, tab_b, idxf, x1h, x2h,
          o1f, o2f,
          idx_v, rows_a, rows_b, sem_a, sem_b,
          l1, l2, xd1, xd2):
        wid = lax.axis_index("s") * _NC + lax.axis_index("c")
        base = wid * per_w
        sbase = wid * s_per_w

        pltpu.sync_copy(x1h.at[pl.ds(sbase, s_per_w)], xd1)
        pltpu.sync_copy(x2h.at[pl.ds(sbase, s_per_w)], xd2)
        pltpu.sync_copy(idxf.at[pl.ds(wid * n_units, n_units)], idx_v)

        lane15 = lax.iota(jnp.int32, 16) == 15

        def start(u, slot):
            pltpu.async_copy(tab_a.at[idx_v.at[u]], rows_a.at[slot], sem_a.at[slot])
            pltpu.async_copy(tab_b.at[idx_v.at[u]], rows_b.at[slot], sem_b.at[slot])

        def compute(u, slot):
            pltpu.make_async_copy(tab_a.at[idx_v.at[u]], rows_a.at[slot], sem_a.at[slot]).wait()
            pltpu.make_async_copy(tab_b.at[idx_v.at[u]], rows_b.at[slot], sem_b.at[slot]).wait()
            s = u // (K1 // _CHUNK)
            lbase = u * _CHUNK
            xv1 = [xd1[s, pl.ds(16 * l, 16)] for l in range(nl)]
            xv2 = [xd2[s, pl.ds(16 * l, 16)] for l in range(nl)]

            def jbody(t, _):
                for r4 in range(8):
                    j = t * 8 + r4
                    acc_a = rows_a[slot, j, pl.ds(0, 16)] * xv1[0]
                    acc_b = rows_b[slot, j, pl.ds(0, 16)] * xv2[0]
                    for l in range(1, nl):
                        sl = pl.ds(16 * l, 16)
                        acc_a = acc_a + rows_a[slot, j, sl] * xv1[l]
                        acc_b = acc_b + rows_b[slot, j, sl] * xv2[l]
                    tgt = jnp.full((16,), lbase + j, jnp.int32)
                    plsc.store_scatter(l1, [tgt], plsc.cumsum(acc_a), mask=lane15)
                    plsc.store_scatter(l2, [tgt], plsc.cumsum(acc_b), mask=lane15)
                return _

            lax.fori_loop(0, _CHUNK // 8, jbody, None)

        for p in range(4):
            start(p, p)

        def gbody(t, _):
            u0 = 4 * t
            for p in range(4):
                compute(u0 + p, p)

                @pl.when(u0 + p + 4 < n_units)
                def _():
                    start(u0 + p + 4, p)

            return _

        lax.fori_loop(0, n_units // 4, gbody, None)

        ng = K1 // 16

        def smax(s, _):
            lb = s * K1
            v1 = [l1[pl.ds(lb + 16 * g, 16)] for g in range(ng)]
            m = v1[0]
            for g in range(1, ng):
                m = jnp.maximum(m, v1[g])
            mv = jnp.full((16,), jnp.max(m), jnp.float32)
            e1 = [jnp.exp(v - mv) for v in v1]
            ssum = e1[0]
            for g in range(1, ng):
                ssum = ssum + e1[g]
            sv = jnp.full((16,), jnp.sum(ssum), jnp.float32)
            o1 = [ev / sv for ev in e1]
            mo = o1[0]
            for g in range(1, ng):
                mo = jnp.maximum(mo, o1[g])
            for g in range(ng):
                l1[pl.ds(lb + 16 * g, 16)] = o1[g]
            mov = jnp.full((16,), jnp.max(mo), jnp.float32)
            v2 = [l2[pl.ds(lb + 16 * g, 16)] for g in range(ng)]
            e2 = [jnp.exp(v - mov) for v in v2]
            ssum2 = e2[0]
            for g in range(1, ng):
                ssum2 = ssum2 + e2[g]
            sv2 = jnp.full((16,), jnp.sum(ssum2), jnp.float32)
            for g in range(ng):
                l2[pl.ds(lb + 16 * g, 16)] = e2[g] / sv2
            return _

        lax.fori_loop(0, s_per_w, smax, None)

        pltpu.sync_copy(l1, o1f.at[pl.ds(base, per_w)])
        pltpu.sync_copy(l2, o2f.at[pl.ds(base, per_w)])

    return k(table_a, table_b, idx_flat.reshape(-1, _CHUNK), x1, x2)


_CP = 128000


def _copy_body(a_ref, b_ref, oa_ref, ob_ref):
    oa_ref[...] = a_ref[...]
    ob_ref[...] = b_ref[...]


def _tc_copy(a, b):
    E = a.shape[0]
    grid = (E // _CP,)
    return pl.pallas_call(
        _copy_body,
        grid=grid,
        in_specs=[
            pl.BlockSpec((_CP,), lambda i: (i,)),
            pl.BlockSpec((_CP,), lambda i: (i,)),
        ],
        out_specs=[
            pl.BlockSpec((_CP,), lambda i: (i,)),
            pl.BlockSpec((_CP,), lambda i: (i,)),
        ],
        out_shape=[
            jax.ShapeDtypeStruct((E,), jnp.float32),
            jax.ShapeDtypeStruct((E,), jnp.float32),
        ],
    )(a, b)


_TB = 32


def _scat_body(u1_ref, u2_ref, idxs_ref, cna_ref, cnb_ref, na_ref, nb_ref, sem):
    D = u1_ref.shape[0] // _TB
    descs = []
    for r in range(_TB):
        tgt = idxs_ref[0, 0, r]
        descs.append(pltpu.make_async_copy(
            u1_ref.at[pl.ds(r * D, D)], na_ref.at[pl.ds(tgt * D, D)], sem))
        descs.append(pltpu.make_async_copy(
            u2_ref.at[pl.ds(r * D, D)], nb_ref.at[pl.ds(tgt * D, D)], sem))
    for d in descs:
        d.start()
    for d in descs:
        d.wait()


def _tc_scatter(u1f, u2f, index3d, cna, cnb):
    B = index3d.shape[0] * _TB
    D = u1f.shape[0] // B
    E = cna.shape[0]
    grid = (B // _TB,)
    return pl.pallas_call(
        _scat_body,
        grid=grid,
        in_specs=[
            pl.BlockSpec((_TB * D,), lambda i: (i,)),
            pl.BlockSpec((_TB * D,), lambda i: (i,)),
            pl.BlockSpec((1, 1, _TB), lambda i: (i, 0, 0), memory_space=pltpu.SMEM),
            pl.BlockSpec(memory_space=pl.ANY),
            pl.BlockSpec(memory_space=pl.ANY),
        ],
        out_specs=[
            pl.BlockSpec(memory_space=pl.ANY),
            pl.BlockSpec(memory_space=pl.ANY),
        ],
        out_shape=[
            jax.ShapeDtypeStruct((E,), jnp.float32),
            jax.ShapeDtypeStruct((E,), jnp.float32),
        ],
        input_output_aliases={3: 0, 4: 1},
        scratch_shapes=[pltpu.SemaphoreType.DMA],
    )(u1f, u2f, index3d, cna, cnb)


def kernel(x1, x2, memory_x1, memory_x2, index, idx):
    B, D = x1.shape
    N = memory_x1.shape[0]
    K1 = idx.shape[1]
    flat = idx.reshape(-1)
    ar = jnp.arange(B, dtype=jnp.int32)
    eff = jnp.max(jnp.where(index[None, :] == index[:, None], ar[None, :], -1), axis=1)
    upd1, upd2 = _sc_upd(memory_x2, memory_x1, x1, x2, index, eff)
    o1f, o2f = _sc_fused(memory_x2, memory_x1, flat, x1, x2)
    cna, cnb = _tc_copy(memory_x1.reshape(-1), memory_x2.reshape(-1))
    na, nb = _tc_scatter(
        upd1.reshape(-1), upd2.reshape(-1), index.reshape(B // _TB, 1, _TB),
        cna, cnb)
    return (o1f.reshape(B, K1, 1), o2f.reshape(B, K1, 1),
            na.reshape(N, D), nb.reshape(N, D))


# speedup vs baseline: 1.2298x; 1.0015x over previous
import functools
import math

import jax
import jax.numpy as jnp
from jax import lax
from jax.experimental import pallas as pl
from jax.experimental.pallas import tpu as pltpu
from jax.experimental.pallas import tpu_sc as plsc

MOMENTUM = 0.5

_NC, _NS = 2, 16
_NW = _NC * _NS
_CHUNK = 64


def _row_normalize(vmem_rows, n_rows, D):
    nv = D // 16

    def body(r, _):
        acc = jnp.zeros((16,), jnp.float32)
        for l in range(nv):
            v = vmem_rows[r, pl.ds(l * 16, 16)]
            acc = acc + v * v
        ss = jnp.sum(acc)
        ssv = jnp.full((16,), ss, jnp.float32)
        y = plsc.bitcast(
            jnp.int32(0x5F3759DF) - (plsc.bitcast(ssv, jnp.int32) >> 1),
            jnp.float32,
        )
        for _i in range(3):
            y = y * (1.5 - 0.5 * ssv * y * y)
        for l in range(nv):
            vmem_rows[r, pl.ds(l * 16, 16)] = vmem_rows[r, pl.ds(l * 16, 16)] * y
        return _

    lax.fori_loop(0, n_rows, body, None)


def _sc_fused(table_a, table_b, idx_flat, x1, x2):
    R = idx_flat.shape[0]
    N, D = table_a.shape
    B = x1.shape[0]
    K1 = R // B
    per_w = R // _NW
    n_units = per_w // _CHUNK
    s_per_w = B // _NW
    nl = D // 16
    mesh = plsc.VectorSubcoreMesh(core_axis_name="c", subcore_axis_name="s")

    @functools.partial(
        pl.kernel,
        out_type=[
            jax.ShapeDtypeStruct((R,), jnp.float32),
            jax.ShapeDtypeStruct((R,), jnp.float32),
        ],
        mesh=mesh,
        scratch_types=[
            pltpu.VMEM((n_units, _CHUNK), jnp.int32),
            pltpu.VMEM((4, _CHUNK, D), jnp.float32),
            pltpu.VMEM((4, _CHUNK, D), jnp.float32),
            pltpu.SemaphoreType.DMA((4,)),
            pltpu.SemaphoreType.DMA((4,)),
            pltpu.VMEM((per_w,), jnp.float32),
            pltpu.VMEM((per_w,), jnp.float32),
            pltpu.VMEM((s_per_w, D), jnp.float32),
            pltpu.VMEM((s_per_w, D), jnp.float32),
        ],
        compiler_params=pltpu.CompilerParams(needs_layout_passes=False),
    )
    def k(tab_a, tab_b, idxf, x1h, x2h,
          o1f, o2f,
          idx_v, rows_a, rows_b, sem_a, sem_b,
          l1, l2, xd1, xd2):
        wid = lax.axis_index("s") * _NC + lax.axis_index("c")
        base = wid * per_w
        sbase = wid * s_per_w

        pltpu.sync_copy(x1h.at[pl.ds(sbase, s_per_w)], xd1)
        pltpu.sync_copy(x2h.at[pl.ds(sbase, s_per_w)], xd2)
        pltpu.sync_copy(idxf.at[pl.ds(wid * n_units, n_units)], idx_v)

        lane15 = lax.iota(jnp.int32, 16) == 15

        def start(u, slot):
            pltpu.async_copy(tab_a.at[idx_v.at[u]], rows_a.at[slot], sem_a.at[slot])
            pltpu.async_copy(tab_b.at[idx_v.at[u]], rows_b.at[slot], sem_b.at[slot])

        def compute(u, slot):
            pltpu.make_async_copy(tab_a.at[idx_v.at[u]], rows_a.at[slot], sem_a.at[slot]).wait()
            pltpu.make_async_copy(tab_b.at[idx_v.at[u]], rows_b.at[slot], sem_b.at[slot]).wait()
            s = u // (K1 // _CHUNK)
            lbase = u * _CHUNK
            xv1 = [xd1[s, pl.ds(16 * l, 16)] for l in range(nl)]
            xv2 = [xd2[s, pl.ds(16 * l, 16)] for l in range(nl)]

            def jbody(t, _):
                for r4 in range(8):
                    j = t * 8 + r4
                    acc_a = rows_a[slot, j, pl.ds(0, 16)] * xv1[0]
                    acc_b = rows_b[slot, j, pl.ds(0, 16)] * xv2[0]
                    for l in range(1, nl):
                        sl = pl.ds(16 * l, 16)
                        acc_a = acc_a + rows_a[slot, j, sl] * xv1[l]
                        acc_b = acc_b + rows_b[slot, j, sl] * xv2[l]
                    tgt = jnp.full((16,), lbase + j, jnp.int32)
                    plsc.store_scatter(l1, [tgt], plsc.cumsum(acc_a), mask=lane15)
                    plsc.store_scatter(l2, [tgt], plsc.cumsum(acc_b), mask=lane15)
                return _

            lax.fori_loop(0, _CHUNK // 8, jbody, None)

        for p in range(4):
            start(p, p)

        def gbody(t, _):
            u0 = 4 * t
            for p in range(4):
                compute(u0 + p, p)

                @pl.when(u0 + p + 4 < n_units)
                def _():
                    start(u0 + p + 4, p)

            return _

        lax.fori_loop(0, n_units // 4, gbody, None)

        ng = K1 // 16

        def smax(s, _):
            lb = s * K1
            v1 = [l1[pl.ds(lb + 16 * g, 16)] for g in range(ng)]
            m = v1[0]
            for g in range(1, ng):
                m = jnp.maximum(m, v1[g])
            mv = jnp.full((16,), jnp.max(m), jnp.float32)
            e1 = [jnp.exp(v - mv) for v in v1]
            ssum = e1[0]
            for g in range(1, ng):
                ssum = ssum + e1[g]
            sv = jnp.full((16,), jnp.sum(ssum), jnp.float32)
            o1 = [ev / sv for ev in e1]
            mo = o1[0]
            for g in range(1, ng):
                mo = jnp.maximum(mo, o1[g])
            for g in range(ng):
                l1[pl.ds(lb + 16 * g, 16)] = o1[g]
            mov = jnp.full((16,), jnp.max(mo), jnp.float32)
            v2 = [l2[pl.ds(lb + 16 * g, 16)] for g in range(ng)]
            e2 = [jnp.exp(v - mov) for v in v2]
            ssum2 = e2[0]
            for g in range(1, ng):
                ssum2 = ssum2 + e2[g]
            sv2 = jnp.full((16,), jnp.sum(ssum2), jnp.float32)
            for g in range(ng):
                l2[pl.ds(lb + 16 * g, 16)] = e2[g] / sv2
            return _

        lax.fori_loop(0, s_per_w, smax, None)

        pltpu.sync_copy(l1, o1f.at[pl.ds(base, per_w)])
        pltpu.sync_copy(l2, o2f.at[pl.ds(base, per_w)])

    return k(table_a, table_b, idx_flat.reshape(-1, _CHUNK), x1, x2)


_CP = 128000


def _copy_body(a_ref, b_ref, oa_ref, ob_ref):
    oa_ref[...] = a_ref[...]
    ob_ref[...] = b_ref[...]


def _tc_copy(a, b):
    E = a.shape[0]
    grid = (E // _CP,)
    return pl.pallas_call(
        _copy_body,
        grid=grid,
        in_specs=[
            pl.BlockSpec((_CP,), lambda i: (i,)),
            pl.BlockSpec((_CP,), lambda i: (i,)),
        ],
        out_specs=[
            pl.BlockSpec((_CP,), lambda i: (i,)),
            pl.BlockSpec((_CP,), lambda i: (i,)),
        ],
        out_shape=[
            jax.ShapeDtypeStruct((E,), jnp.float32),
            jax.ShapeDtypeStruct((E,), jnp.float32),
        ],
    )(a, b)


_TB = 32


def _scat_body(m1_ref, m2_ref, x1_ref, x2_ref, idxs_ref, effs_ref,
               cna_ref, cnb_ref, na_ref, nb_ref,
               g1, g2, xs1, xs2, u1, u2, sem):
    D = g1.shape[1]
    gd = []
    for r in range(_TB):
        tgt = idxs_ref[0, 0, r]
        src = effs_ref[0, 0, r]
        gd.append(pltpu.make_async_copy(
            m1_ref.at[pl.ds(tgt * D, D)], g1.at[r], sem))
        gd.append(pltpu.make_async_copy(
            m2_ref.at[pl.ds(tgt * D, D)], g2.at[r], sem))
        gd.append(pltpu.make_async_copy(
            x1_ref.at[pl.ds(src * D, D)], xs1.at[r], sem))
        gd.append(pltpu.make_async_copy(
            x2_ref.at[pl.ds(src * D, D)], xs2.at[r], sem))
    for d in gd:
        d.start()
    for d in gd:
        d.wait()

    def upd(g_ref, x_ref, u_ref):
        pos = g_ref[...] * MOMENTUM + x_ref[...] * (1.0 - MOMENTUM)
        inv = 1.0 / jnp.sqrt(jnp.sum(pos * pos, axis=1, keepdims=True))
        u_ref[...] = pos * inv

    upd(g1, xs1, u1)
    upd(g2, xs2, u2)

    sd = []
    for r in range(_TB):
        tgt = idxs_ref[0, 0, r]
        sd.append(pltpu.make_async_copy(u1.at[r], na_ref.at[pl.ds(tgt * D, D)], sem))
        sd.append(pltpu.make_async_copy(u2.at[r], nb_ref.at[pl.ds(tgt * D, D)], sem))
    for d in sd:
        d.start()
    for d in sd:
        d.wait()


def _tc_scatter(m1f, m2f, x1f, x2f, index3d, eff3d, cna, cnb, D):
    B = index3d.shape[0] * _TB
    E = cna.shape[0]
    grid = (B // _TB,)
    return pl.pallas_call(
        _scat_body,
        grid=grid,
        in_specs=[
            pl.BlockSpec(memory_space=pl.ANY),
            pl.BlockSpec(memory_space=pl.ANY),
            pl.BlockSpec(memory_space=pl.ANY),
            pl.BlockSpec(memory_space=pl.ANY),
            pl.BlockSpec((1, 1, _TB), lambda i: (i, 0, 0), memory_space=pltpu.SMEM),
            pl.BlockSpec((1, 1, _TB), lambda i: (i, 0, 0), memory_space=pltpu.SMEM),
            pl.BlockSpec(memory_space=pl.ANY),
            pl.BlockSpec(memory_space=pl.ANY),
        ],
        out_specs=[
            pl.BlockSpec(memory_space=pl.ANY),
            pl.BlockSpec(memory_space=pl.ANY),
        ],
        out_shape=[
            jax.ShapeDtypeStruct((E,), jnp.float32),
            jax.ShapeDtypeStruct((E,), jnp.float32),
        ],
        input_output_aliases={6: 0, 7: 1},
        scratch_shapes=[
            pltpu.VMEM((_TB, 128), jnp.float32),
            pltpu.VMEM((_TB, 128), jnp.float32),
            pltpu.VMEM((_TB, 128), jnp.float32),
            pltpu.VMEM((_TB, 128), jnp.float32),
            pltpu.VMEM((_TB, 128), jnp.float32),
            pltpu.VMEM((_TB, 128), jnp.float32),
            pltpu.SemaphoreType.DMA,
        ],
    )(m1f, m2f, x1f, x2f, index3d, eff3d, cna, cnb)


def kernel(x1, x2, memory_x1, memory_x2, index, idx):
    B, D = x1.shape
    N = memory_x1.shape[0]
    K1 = idx.shape[1]
    flat = idx.reshape(-1)
    ar = jnp.arange(B, dtype=jnp.int32)
    eff = jnp.max(jnp.where(index[None, :] == index[:, None], ar[None, :], -1), axis=1)
    o1f, o2f = _sc_fused(memory_x2, memory_x1, flat, x1, x2)
    cna, cnb = _tc_copy(memory_x1.reshape(-1), memory_x2.reshape(-1))
    na, nb = _tc_scatter(
        memory_x1.reshape(-1), memory_x2.reshape(-1),
        x1.reshape(-1), x2.reshape(-1),
        index.reshape(B // _TB, 1, _TB), eff.reshape(B // _TB, 1, _TB),
        cna, cnb, D)
    return (o1f.reshape(B, K1, 1), o2f.reshape(B, K1, 1),
            na.reshape(N, D), nb.reshape(N, D))
